# baseline (speedup 1.0000x reference)
import functools

import jax
import jax.numpy as jnp
from jax import lax
from jax.experimental import pallas as pl
from jax.experimental.pallas import tpu as pltpu
from jax.experimental.pallas import tpu_sc as plsc

VOCAB = 100000
DIM = 64
MAXLEN = 200
BATCH = 1024

NC = 2
NS = 16
NW = NC * NS
ROWS_PER_W = BATCH // NW

IDX_SPLIT = 2
IDX_CHUNK = MAXLEN // IDX_SPLIT

NBUF = 4


def _make_kernel():
  mesh = plsc.VectorSubcoreMesh(core_axis_name="c", subcore_axis_name="s")

  @functools.partial(
      pl.kernel,
      out_type=jax.ShapeDtypeStruct((BATCH, MAXLEN, DIM), jnp.float32),
      mesh=mesh,
      scratch_types=[
          pltpu.VMEM((ROWS_PER_W, IDX_SPLIT, IDX_CHUNK), jnp.int32),
          pltpu.VMEM_SHARED((MAXLEN, DIM), jnp.float32),
      ]
      + [pltpu.VMEM((MAXLEN, DIM), jnp.float32) for _ in range(NBUF)]
      + [pltpu.SemaphoreType.DMA for _ in range(2 * NBUF)],
      compiler_params=pltpu.CompilerParams(use_tc_tiling_on_sc=False),
  )
  def tok_pos_embed(idx_hbm, tok_hbm, pos_hbm, out_hbm, idx_all, pos_sh,
                    *bufs_and_sems):
    bufs = bufs_and_sems[:NBUF]
    gsem = bufs_and_sems[NBUF:2 * NBUF]
    osem = bufs_and_sems[2 * NBUF:]
    cid = lax.axis_index("c")
    sid = lax.axis_index("s")
    wid = sid * NC + cid
    row0 = wid * ROWS_PER_W

    @pl.when(sid == 0)
    def _():
      pltpu.sync_copy(pos_hbm, pos_sh)
    pltpu.sync_copy(idx_hbm.at[pl.ds(row0, ROWS_PER_W)], idx_all)
    plsc.subcore_barrier()

    def fire_row(r):
      p = r % NBUF
      pltpu.sync_copy(pos_sh, bufs[p])
      return [
          pltpu.async_copy(
              tok_hbm.at[idx_all.at[r, j]],
              bufs[p].at[pl.ds(j * IDX_CHUNK, IDX_CHUNK)],
              gsem[p],
              add=True,
          )
          for j in range(IDX_SPLIT)
      ]

    gathers = {}
    outs = {}
    for r in range(min(NBUF, ROWS_PER_W)):
      gathers[r] = fire_row(r)

    for r in range(ROWS_PER_W):
      p = r % NBUF
      for d in gathers.pop(r):
        d.wait()
      outs[r] = pltpu.async_copy(bufs[p], out_hbm.at[row0 + r], osem[p])
      if r + NBUF < ROWS_PER_W:
        outs.pop(r).wait()
        gathers[r + NBUF] = fire_row(r + NBUF)

    for r, d in outs.items():
      d.wait()

  return tok_pos_embed


_KERNEL = _make_kernel()


def kernel(inputs, token_table, pos_table):
  idx = inputs.astype(jnp.int32).reshape(BATCH, IDX_SPLIT, IDX_CHUNK)
  return _KERNEL(idx, token_table, pos_table)
